# Initial kernel scaffold; baseline (speedup 1.0000x reference)
import jax
import jax.numpy as jnp
from jax.experimental import pallas as pl


def kernel(hidden_state, schema_indices):
    raise NotImplementedError("write your pallas kernel here")


# speedup vs baseline: 1.1685x; 1.1685x over previous
import functools

import jax
import jax.numpy as jnp
from jax import lax
from jax.experimental import pallas as pl
from jax.experimental.pallas import tpu as pltpu
from jax.experimental.pallas import tpu_sc as plsc

_D = 4096
_B_FIELDS = 256
_NC = 2
_NS = 16
_NW = _NC * _NS
_ROWS_PER_W = _B_FIELDS // _NW

_mesh = plsc.VectorSubcoreMesh(core_axis_name="c", subcore_axis_name="s")


@functools.partial(
    pl.kernel,
    out_type=[
        jax.ShapeDtypeStruct((1, _D), jnp.float32),
        jax.ShapeDtypeStruct((_B_FIELDS, _D), jnp.float32),
    ],
    mesh=_mesh,
    scratch_types=[
        pltpu.VMEM((_ROWS_PER_W,), jnp.int32),
        pltpu.VMEM((_ROWS_PER_W, _D), jnp.float32),
        pltpu.VMEM((1,), jnp.int32),
        pltpu.VMEM((1, _D), jnp.float32),
        pltpu.SemaphoreType.DMA,
    ],
)
def _sc_gather(table_hbm, idxf_hbm, idxp_hbm, pc_hbm, fields_hbm,
               idx_v, rows_v, idxp_v, row_pc, sem):
    wid = lax.axis_index("s") * _NC + lax.axis_index("c")
    base = wid * _ROWS_PER_W
    pltpu.sync_copy(idxf_hbm.at[pl.ds(base, _ROWS_PER_W)], idx_v)
    pltpu.async_copy(table_hbm.at[idx_v], rows_v, sem).wait()
    pltpu.sync_copy(rows_v, fields_hbm.at[pl.ds(base, _ROWS_PER_W)])

    @pl.when(wid == 0)
    def _():
        pltpu.sync_copy(idxp_hbm, idxp_v)
        pltpu.async_copy(table_hbm.at[idxp_v], row_pc, sem).wait()
        pltpu.sync_copy(row_pc, pc_hbm)


def kernel(hidden_state, schema_indices):
    table = hidden_state[0]
    idx_fields = schema_indices[1:]
    idx_pc = schema_indices[0:1]
    pc_emb, field_embs = _sc_gather(table, idx_fields, idx_pc)
    return (pc_emb, field_embs)


# speedup vs baseline: 1.2224x; 1.0461x over previous
import functools

import jax
import jax.numpy as jnp
from jax import lax
from jax.experimental import pallas as pl
from jax.experimental.pallas import tpu as pltpu
from jax.experimental.pallas import tpu_sc as plsc

_D = 4096
_B_FIELDS = 256
_NC = 2
_NS = 16
_NW = _NC * _NS
_ROWS_PER_W = _B_FIELDS // _NW

_mesh = plsc.VectorSubcoreMesh(core_axis_name="c", subcore_axis_name="s")


_CHUNK = 4


@functools.partial(
    pl.kernel,
    out_type=[
        jax.ShapeDtypeStruct((1, _D), jnp.float32),
        jax.ShapeDtypeStruct((_B_FIELDS, _D), jnp.float32),
    ],
    mesh=_mesh,
    scratch_types=[
        pltpu.VMEM((_CHUNK,), jnp.int32),
        pltpu.VMEM((_CHUNK,), jnp.int32),
        pltpu.VMEM((_CHUNK, _D), jnp.float32),
        pltpu.VMEM((_CHUNK, _D), jnp.float32),
        pltpu.VMEM((1,), jnp.int32),
        pltpu.VMEM((1, _D), jnp.float32),
        pltpu.SemaphoreType.DMA,
        pltpu.SemaphoreType.DMA,
        pltpu.SemaphoreType.DMA,
        pltpu.SemaphoreType.DMA,
    ],
)
def _sc_gather(table_hbm, idxf_hbm, idxp_hbm, pc_hbm, fields_hbm,
               idx_a, idx_b, rows_a, rows_b, idxp_v, row_pc,
               sem_a, sem_b, sem_p, sem_s):
    wid = lax.axis_index("s") * _NC + lax.axis_index("c")
    base = wid * _ROWS_PER_W
    is_w0 = wid == 0

    ia = pltpu.async_copy(idxf_hbm.at[wid, 0], idx_a, sem_a)
    ib = pltpu.async_copy(idxf_hbm.at[wid, 1], idx_b, sem_b)

    @pl.when(is_w0)
    def _():
        pltpu.sync_copy(idxp_hbm, idxp_v)

    ia.wait()
    ga = pltpu.async_copy(table_hbm.at[idx_a], rows_a, sem_a)
    ib.wait()
    gb = pltpu.async_copy(table_hbm.at[idx_b], rows_b, sem_b)

    @pl.when(is_w0)
    def _():
        pltpu.async_copy(table_hbm.at[idxp_v], row_pc, sem_p)

    ga.wait()
    sa = pltpu.async_copy(rows_a, fields_hbm.at[pl.ds(base, _CHUNK)], sem_s)
    gb.wait()
    sb = pltpu.async_copy(rows_b, fields_hbm.at[pl.ds(base + _CHUNK, _CHUNK)], sem_s)

    @pl.when(is_w0)
    def _():
        pltpu.make_async_copy(table_hbm.at[idxp_v], row_pc, sem_p).wait()
        pltpu.async_copy(row_pc, pc_hbm, sem_p).wait()

    sa.wait()
    sb.wait()


def kernel(hidden_state, schema_indices):
    table = hidden_state[0]
    idx_fields = schema_indices[1:].reshape(_NW, _ROWS_PER_W // _CHUNK, _CHUNK)
    idx_pc = schema_indices[0:1]
    pc_emb, field_embs = _sc_gather(table, idx_fields, idx_pc)
    return (pc_emb, field_embs)
